# baseline (device time: 37900 ns/iter reference)
import jax
import jax.numpy as jnp
from jax import lax
from jax.experimental import pallas as pl
from jax.experimental.pallas import tpu as pltpu

N_DEV = 8
S = 4


def _gelu(y):
    c = 0.7978845608028654
    return 0.5 * y * (1.0 + jnp.tanh(c * (y + 0.044715 * y * y * y)))


def kernel(x, w_mat):
    m, _ = x.shape
    _, n = w_mat.shape
    cm = m // N_DEV
    hm = cm // S

    def body(
        x_ref,
        w_ref,
        out_ref,
        red_ref,
        pbf_ref,
        rs_buf,
        g_buf,
        ag_buf,
        rs_send_sems,
        rs_recv_sems,
        ag_send_sems,
        ag_recv_sems,
    ):
        d = lax.axis_index("i")

        bar = pltpu.get_barrier_semaphore()

        def _bar_round(k):
            nbr = lax.bitwise_xor(d, k)
            pl.semaphore_signal(
                bar,
                inc=1,
                device_id=(nbr,),
                device_id_type=pl.DeviceIdType.MESH,
            )

        _bar_round(1)
        wb = w_ref[...].astype(jnp.bfloat16)
        for t in range(1, N_DEV):
            tgt = lax.rem(d + t, N_DEV)
            xc = x_ref[pl.ds(tgt * cm, cm), :].astype(jnp.bfloat16)
            pc = jnp.dot(xc, wb, preferred_element_type=jnp.float32)
            pbf_ref[t - 1] = pc.astype(jnp.bfloat16)
        xc = x_ref[pl.ds(d * cm, cm), :].astype(jnp.bfloat16)
        red_ref[...] = jnp.dot(xc, wb, preferred_element_type=jnp.float32)

        pl.semaphore_wait(bar, 1)
        _bar_round(3)
        pl.semaphore_wait(bar, 1)
        _bar_round(4)
        pl.semaphore_wait(bar, 1)

        rs_sends = []
        for t in range(1, N_DEV):
            tgt = lax.rem(d + t, N_DEV)
            slot = N_DEV - 1 - t
            for a in range(S):
                rdma = pltpu.make_async_remote_copy(
                    src_ref=pbf_ref.at[t - 1, pl.ds(a * hm, hm), :],
                    dst_ref=rs_buf.at[slot * S + a],
                    send_sem=rs_send_sems.at[(t - 1) * S + a],
                    recv_sem=rs_recv_sems.at[slot * S + a],
                    device_id=(tgt,),
                    device_id_type=pl.DeviceIdType.MESH,
                )
                rdma.start()
                rs_sends.append(rdma)

        ag_sends = []
        for a in range(S):
            for slot in range(N_DEV - 1):
                recv = pltpu.make_async_remote_copy(
                    src_ref=rs_buf.at[slot * S + a],
                    dst_ref=rs_buf.at[slot * S + a],
                    send_sem=rs_send_sems.at[0],
                    recv_sem=rs_recv_sems.at[slot * S + a],
                    device_id=(d,),
                    device_id_type=pl.DeviceIdType.MESH,
                )
                recv.wait_recv()
                red_ref[pl.ds(a * hm, hm), :] += rs_buf[slot * S + a].astype(
                    jnp.float32
                )

            g = _gelu(red_ref[pl.ds(a * hm, hm), :])
            out_ref[pl.ds(d * cm + a * hm, hm), :] = g
            g_buf[a] = g.astype(jnp.bfloat16)

            for t in range(1, N_DEV):
                tgt = lax.rem(d + t, N_DEV)
                slot = N_DEV - 1 - t
                rdma = pltpu.make_async_remote_copy(
                    src_ref=g_buf.at[a],
                    dst_ref=ag_buf.at[slot * S + a],
                    send_sem=ag_send_sems.at[(t - 1) * S + a],
                    recv_sem=ag_recv_sems.at[slot * S + a],
                    device_id=(tgt,),
                    device_id_type=pl.DeviceIdType.MESH,
                )
                rdma.start()
                ag_sends.append(rdma)

        for a in range(S):
            for slot in range(N_DEV - 1):
                recv = pltpu.make_async_remote_copy(
                    src_ref=ag_buf.at[slot * S + a],
                    dst_ref=ag_buf.at[slot * S + a],
                    send_sem=ag_send_sems.at[0],
                    recv_sem=ag_recv_sems.at[slot * S + a],
                    device_id=(d,),
                    device_id_type=pl.DeviceIdType.MESH,
                )
                recv.wait_recv()
                ci = lax.rem(d + slot + 1, N_DEV)
                out_ref[pl.ds(ci * cm + a * hm, hm), :] = ag_buf[
                    slot * S + a
                ].astype(jnp.float32)

        for rdma in rs_sends:
            rdma.wait_send()
        for rdma in ag_sends:
            rdma.wait_send()

    nsub = (N_DEV - 1) * S
    return pl.pallas_call(
        body,
        out_shape=jax.ShapeDtypeStruct((m, n), jnp.float32),
        in_specs=[
            pl.BlockSpec(memory_space=pltpu.VMEM),
            pl.BlockSpec(memory_space=pltpu.VMEM),
        ],
        out_specs=pl.BlockSpec(memory_space=pltpu.VMEM),
        scratch_shapes=[
            pltpu.VMEM((cm, n), jnp.float32),
            pltpu.VMEM((N_DEV - 1, cm, n), jnp.bfloat16),
            pltpu.VMEM((nsub, hm, n), jnp.bfloat16),
            pltpu.VMEM((S, hm, n), jnp.bfloat16),
            pltpu.VMEM((nsub, hm, n), jnp.bfloat16),
            pltpu.SemaphoreType.DMA((nsub,)),
            pltpu.SemaphoreType.DMA((nsub,)),
            pltpu.SemaphoreType.DMA((nsub,)),
            pltpu.SemaphoreType.DMA((nsub,)),
        ],
        compiler_params=pltpu.CompilerParams(collective_id=0),
    )(x, w_mat)


# device time: 37581 ns/iter; 1.0085x vs baseline; 1.0085x over previous
import jax
import jax.numpy as jnp
from jax import lax
from jax.experimental import pallas as pl
from jax.experimental.pallas import tpu as pltpu

N_DEV = 8
S = 2


def _gelu(y):
    c = 0.7978845608028654
    return 0.5 * y * (1.0 + jnp.tanh(c * (y + 0.044715 * y * y * y)))


def kernel(x, w_mat):
    m, _ = x.shape
    _, n = w_mat.shape
    cm = m // N_DEV
    hm = cm // S

    def body(
        x_ref,
        w_ref,
        out_ref,
        red_ref,
        pbf_ref,
        rs_buf,
        g_buf,
        ag_buf,
        rs_send_sems,
        rs_recv_sems,
        ag_send_sems,
        ag_recv_sems,
    ):
        d = lax.axis_index("i")

        bar = pltpu.get_barrier_semaphore()

        def _bar_round(k):
            nbr = lax.bitwise_xor(d, k)
            pl.semaphore_signal(
                bar,
                inc=1,
                device_id=(nbr,),
                device_id_type=pl.DeviceIdType.MESH,
            )

        _bar_round(1)
        wb = w_ref[...].astype(jnp.bfloat16)
        for t in range(1, N_DEV):
            tgt = lax.rem(d + t, N_DEV)
            xc = x_ref[pl.ds(tgt * cm, cm), :].astype(jnp.bfloat16)
            pc = jnp.dot(xc, wb, preferred_element_type=jnp.float32)
            pbf_ref[t - 1] = pc.astype(jnp.bfloat16)
        xc = x_ref[pl.ds(d * cm, cm), :].astype(jnp.bfloat16)
        red_ref[...] = jnp.dot(xc, wb, preferred_element_type=jnp.float32)

        pl.semaphore_wait(bar, 1)
        _bar_round(3)
        pl.semaphore_wait(bar, 1)
        _bar_round(4)
        pl.semaphore_wait(bar, 1)

        rs_sends = []
        for t in range(1, N_DEV):
            tgt = lax.rem(d + t, N_DEV)
            slot = N_DEV - 1 - t
            for a in range(S):
                rdma = pltpu.make_async_remote_copy(
                    src_ref=pbf_ref.at[t - 1, pl.ds(a * hm, hm), :],
                    dst_ref=rs_buf.at[slot * S + a],
                    send_sem=rs_send_sems.at[(t - 1) * S + a],
                    recv_sem=rs_recv_sems.at[slot * S + a],
                    device_id=(tgt,),
                    device_id_type=pl.DeviceIdType.MESH,
                )
                rdma.start()
                rs_sends.append(rdma)

        ag_sends = []
        for a in range(S):
            for slot in range(N_DEV - 1):
                recv = pltpu.make_async_remote_copy(
                    src_ref=rs_buf.at[slot * S + a],
                    dst_ref=rs_buf.at[slot * S + a],
                    send_sem=rs_send_sems.at[0],
                    recv_sem=rs_recv_sems.at[slot * S + a],
                    device_id=(d,),
                    device_id_type=pl.DeviceIdType.MESH,
                )
                recv.wait_recv()
                red_ref[pl.ds(a * hm, hm), :] += rs_buf[slot * S + a].astype(
                    jnp.float32
                )

            g = _gelu(red_ref[pl.ds(a * hm, hm), :])
            out_ref[pl.ds(d * cm + a * hm, hm), :] = g
            g_buf[a] = g.astype(jnp.bfloat16)

            for t in range(1, N_DEV):
                tgt = lax.rem(d + t, N_DEV)
                slot = N_DEV - 1 - t
                rdma = pltpu.make_async_remote_copy(
                    src_ref=g_buf.at[a],
                    dst_ref=ag_buf.at[slot * S + a],
                    send_sem=ag_send_sems.at[(t - 1) * S + a],
                    recv_sem=ag_recv_sems.at[slot * S + a],
                    device_id=(tgt,),
                    device_id_type=pl.DeviceIdType.MESH,
                )
                rdma.start()
                ag_sends.append(rdma)

        for a in range(S):
            for slot in range(N_DEV - 1):
                recv = pltpu.make_async_remote_copy(
                    src_ref=ag_buf.at[slot * S + a],
                    dst_ref=ag_buf.at[slot * S + a],
                    send_sem=ag_send_sems.at[0],
                    recv_sem=ag_recv_sems.at[slot * S + a],
                    device_id=(d,),
                    device_id_type=pl.DeviceIdType.MESH,
                )
                recv.wait_recv()
                ci = lax.rem(d + slot + 1, N_DEV)
                out_ref[pl.ds(ci * cm + a * hm, hm), :] = ag_buf[
                    slot * S + a
                ].astype(jnp.float32)

        for rdma in rs_sends:
            rdma.wait_send()
        for rdma in ag_sends:
            rdma.wait_send()

    nsub = (N_DEV - 1) * S
    return pl.pallas_call(
        body,
        out_shape=jax.ShapeDtypeStruct((m, n), jnp.float32),
        in_specs=[
            pl.BlockSpec(memory_space=pltpu.VMEM),
            pl.BlockSpec(memory_space=pltpu.VMEM),
        ],
        out_specs=pl.BlockSpec(memory_space=pltpu.VMEM),
        scratch_shapes=[
            pltpu.VMEM((cm, n), jnp.float32),
            pltpu.VMEM((N_DEV - 1, cm, n), jnp.bfloat16),
            pltpu.VMEM((nsub, hm, n), jnp.bfloat16),
            pltpu.VMEM((S, hm, n), jnp.bfloat16),
            pltpu.VMEM((nsub, hm, n), jnp.bfloat16),
            pltpu.SemaphoreType.DMA((nsub,)),
            pltpu.SemaphoreType.DMA((nsub,)),
            pltpu.SemaphoreType.DMA((nsub,)),
            pltpu.SemaphoreType.DMA((nsub,)),
        ],
        compiler_params=pltpu.CompilerParams(collective_id=0),
    )(x, w_mat)


# device time: 36823 ns/iter; 1.0292x vs baseline; 1.0206x over previous
import jax
import jax.numpy as jnp
from jax import lax
from jax.experimental import pallas as pl
from jax.experimental.pallas import tpu as pltpu

N_DEV = 8
S = 2


def _gelu(y):
    c = 0.7978845608028654
    return 0.5 * y * (1.0 + jnp.tanh(c * (y + 0.044715 * y * y * y)))


def kernel(x, w_mat):
    m, _ = x.shape
    _, n = w_mat.shape
    cm = m // N_DEV
    hm = cm // S

    def body(
        x_ref,
        w_ref,
        out_ref,
        red_ref,
        pbf_ref,
        rs_buf,
        rs_send_sems,
        rs_recv_sems,
        ag_send_sems,
        ag_recv_sems,
    ):
        d = lax.axis_index("i")

        bar = pltpu.get_barrier_semaphore()

        def _bar_signal(k):
            nbr = lax.bitwise_xor(d, k)
            pl.semaphore_signal(
                bar,
                inc=1,
                device_id=(nbr,),
                device_id_type=pl.DeviceIdType.MESH,
            )

        _bar_signal(1)
        wb = w_ref[...].astype(jnp.bfloat16)
        for t in range(1, N_DEV):
            tgt = lax.rem(d + t, N_DEV)
            xc = x_ref[pl.ds(tgt * cm, cm), :].astype(jnp.bfloat16)
            pc = jnp.dot(xc, wb, preferred_element_type=jnp.float32)
            pbf_ref[t - 1] = pc.astype(jnp.bfloat16)
        xc = x_ref[pl.ds(d * cm, cm), :].astype(jnp.bfloat16)
        red_ref[...] = jnp.dot(xc, wb, preferred_element_type=jnp.float32)

        pl.semaphore_wait(bar, 1)
        _bar_signal(3)
        pl.semaphore_wait(bar, 1)
        _bar_signal(4)
        pl.semaphore_wait(bar, 1)

        rs_sends = []
        for t in range(1, N_DEV):
            tgt = lax.rem(d + t, N_DEV)
            slot = N_DEV - 1 - t
            for a in range(S):
                rdma = pltpu.make_async_remote_copy(
                    src_ref=pbf_ref.at[t - 1, pl.ds(a * hm, hm), :],
                    dst_ref=rs_buf.at[slot * S + a],
                    send_sem=rs_send_sems.at[(t - 1) * S + a],
                    recv_sem=rs_recv_sems.at[slot * S + a],
                    device_id=(tgt,),
                    device_id_type=pl.DeviceIdType.MESH,
                )
                rdma.start()
                rs_sends.append(rdma)

        ag_sends = []
        for a in range(S):
            for slot in range(N_DEV - 1):
                recv = pltpu.make_async_remote_copy(
                    src_ref=rs_buf.at[slot * S + a],
                    dst_ref=rs_buf.at[slot * S + a],
                    send_sem=rs_send_sems.at[0],
                    recv_sem=rs_recv_sems.at[slot * S + a],
                    device_id=(d,),
                    device_id_type=pl.DeviceIdType.MESH,
                )
                recv.wait_recv()
                red_ref[pl.ds(a * hm, hm), :] += rs_buf[slot * S + a].astype(
                    jnp.float32
                )

            rows = pl.ds(d * cm + a * hm, hm)
            out_ref[rows, :] = _gelu(red_ref[pl.ds(a * hm, hm), :]).astype(
                jnp.bfloat16
            )

            for t in range(1, N_DEV):
                tgt = lax.rem(d + t, N_DEV)
                rdma = pltpu.make_async_remote_copy(
                    src_ref=out_ref.at[rows, :],
                    dst_ref=out_ref.at[rows, :],
                    send_sem=ag_send_sems.at[(t - 1) * S + a],
                    recv_sem=ag_recv_sems.at[(N_DEV - 1 - t) * S + a],
                    device_id=(tgt,),
                    device_id_type=pl.DeviceIdType.MESH,
                )
                rdma.start()
                ag_sends.append(rdma)

        for i in range((N_DEV - 1) * S):
            recv = pltpu.make_async_remote_copy(
                src_ref=rs_buf.at[0],
                dst_ref=rs_buf.at[0],
                send_sem=ag_send_sems.at[0],
                recv_sem=ag_recv_sems.at[i],
                device_id=(d,),
                device_id_type=pl.DeviceIdType.MESH,
            )
            recv.wait_recv()

        for rdma in rs_sends:
            rdma.wait_send()
        for rdma in ag_sends:
            rdma.wait_send()

    nsub = (N_DEV - 1) * S
    return pl.pallas_call(
        body,
        out_shape=jax.ShapeDtypeStruct((m, n), jnp.bfloat16),
        in_specs=[
            pl.BlockSpec(memory_space=pltpu.VMEM),
            pl.BlockSpec(memory_space=pltpu.VMEM),
        ],
        out_specs=pl.BlockSpec(memory_space=pltpu.VMEM),
        scratch_shapes=[
            pltpu.VMEM((cm, n), jnp.float32),
            pltpu.VMEM((N_DEV - 1, cm, n), jnp.bfloat16),
            pltpu.VMEM((nsub, hm, n), jnp.bfloat16),
            pltpu.SemaphoreType.DMA((nsub,)),
            pltpu.SemaphoreType.DMA((nsub,)),
            pltpu.SemaphoreType.DMA((nsub,)),
            pltpu.SemaphoreType.DMA((nsub,)),
        ],
        compiler_params=pltpu.CompilerParams(collective_id=0),
    )(x, w_mat)


# device time: 36213 ns/iter; 1.0466x vs baseline; 1.0168x over previous
import jax
import jax.numpy as jnp
from jax import lax
from jax.experimental import pallas as pl
from jax.experimental.pallas import tpu as pltpu

N_DEV = 8
S = 2


def _gelu(y):
    c = 0.7978845608028654
    return 0.5 * y * (1.0 + jnp.tanh(c * (y + 0.044715 * y * y * y)))


def kernel(x, w_mat):
    m, _ = x.shape
    _, n = w_mat.shape
    cm = m // N_DEV
    hm = cm // S

    def body(
        x_ref,
        w_ref,
        out_ref,
        red_ref,
        pbf_ref,
        rs_buf,
        rs_send_sems,
        rs_recv_sems,
        ag_send_sems,
        ag_recv_sems,
    ):
        d = lax.axis_index("i")

        bar = pltpu.get_barrier_semaphore()
        for k in range(1, N_DEV):
            pl.semaphore_signal(
                bar,
                inc=1,
                device_id=(lax.bitwise_xor(d, k),),
                device_id_type=pl.DeviceIdType.MESH,
            )

        wb = w_ref[...].astype(jnp.bfloat16)
        for t in range(1, N_DEV):
            tgt = lax.rem(d + t, N_DEV)
            xc = x_ref[pl.ds(tgt * cm, cm), :].astype(jnp.bfloat16)
            pc = jnp.dot(xc, wb, preferred_element_type=jnp.float32)
            pbf_ref[t - 1] = pc.astype(jnp.bfloat16)
        xc = x_ref[pl.ds(d * cm, cm), :].astype(jnp.bfloat16)
        red_ref[...] = jnp.dot(xc, wb, preferred_element_type=jnp.float32)

        pl.semaphore_wait(bar, N_DEV - 1)

        rs_sends = []
        for t in range(1, N_DEV):
            tgt = lax.rem(d + t, N_DEV)
            slot = N_DEV - 1 - t
            for a in range(S):
                rdma = pltpu.make_async_remote_copy(
                    src_ref=pbf_ref.at[t - 1, pl.ds(a * hm, hm), :],
                    dst_ref=rs_buf.at[slot * S + a],
                    send_sem=rs_send_sems.at[(t - 1) * S + a],
                    recv_sem=rs_recv_sems.at[slot * S + a],
                    device_id=(tgt,),
                    device_id_type=pl.DeviceIdType.MESH,
                )
                rdma.start()
                rs_sends.append(rdma)

        ag_sends = []
        for a in range(S):
            for slot in range(N_DEV - 1):
                recv = pltpu.make_async_remote_copy(
                    src_ref=rs_buf.at[slot * S + a],
                    dst_ref=rs_buf.at[slot * S + a],
                    send_sem=rs_send_sems.at[0],
                    recv_sem=rs_recv_sems.at[slot * S + a],
                    device_id=(d,),
                    device_id_type=pl.DeviceIdType.MESH,
                )
                recv.wait_recv()
                red_ref[pl.ds(a * hm, hm), :] += rs_buf[slot * S + a].astype(
                    jnp.float32
                )

            rows = pl.ds(d * cm + a * hm, hm)
            out_ref[rows, :] = _gelu(red_ref[pl.ds(a * hm, hm), :]).astype(
                jnp.bfloat16
            )

            for t in range(1, N_DEV):
                tgt = lax.rem(d + t, N_DEV)
                rdma = pltpu.make_async_remote_copy(
                    src_ref=out_ref.at[rows, :],
                    dst_ref=out_ref.at[rows, :],
                    send_sem=ag_send_sems.at[(t - 1) * S + a],
                    recv_sem=ag_recv_sems.at[(N_DEV - 1 - t) * S + a],
                    device_id=(tgt,),
                    device_id_type=pl.DeviceIdType.MESH,
                )
                rdma.start()
                ag_sends.append(rdma)

        for i in range((N_DEV - 1) * S):
            recv = pltpu.make_async_remote_copy(
                src_ref=rs_buf.at[0],
                dst_ref=rs_buf.at[0],
                send_sem=ag_send_sems.at[0],
                recv_sem=ag_recv_sems.at[i],
                device_id=(d,),
                device_id_type=pl.DeviceIdType.MESH,
            )
            recv.wait_recv()

        for rdma in rs_sends:
            rdma.wait_send()
        for rdma in ag_sends:
            rdma.wait_send()

    nsub = (N_DEV - 1) * S
    return pl.pallas_call(
        body,
        out_shape=jax.ShapeDtypeStruct((m, n), jnp.bfloat16),
        in_specs=[
            pl.BlockSpec(memory_space=pltpu.VMEM),
            pl.BlockSpec(memory_space=pltpu.VMEM),
        ],
        out_specs=pl.BlockSpec(memory_space=pltpu.VMEM),
        scratch_shapes=[
            pltpu.VMEM((cm, n), jnp.float32),
            pltpu.VMEM((N_DEV - 1, cm, n), jnp.bfloat16),
            pltpu.VMEM((nsub, hm, n), jnp.bfloat16),
            pltpu.SemaphoreType.DMA((nsub,)),
            pltpu.SemaphoreType.DMA((nsub,)),
            pltpu.SemaphoreType.DMA((nsub,)),
            pltpu.SemaphoreType.DMA((nsub,)),
        ],
        compiler_params=pltpu.CompilerParams(collective_id=0),
    )(x, w_mat)
